# baseline (speedup 1.0000x reference)
import jax
import jax.numpy as jnp
from jax.experimental import pallas as pl
from jax.experimental.pallas import tpu as pltpu


def _support_body(feat_ref, w_ref, out_ref):
    out_ref[...] = jnp.dot(
        feat_ref[...], w_ref[...], preferred_element_type=jnp.float32
    )


def _agg_body(adj_ref, sup_ref, feat_ref, b_ref, out_ref):
    acc = jnp.dot(adj_ref[...], sup_ref[...], preferred_element_type=jnp.float32)
    out_ref[...] = jnp.maximum(acc + b_ref[...], 0.0) + feat_ref[...]


def kernel(features, adj_norm, weight, bias):
    n, d = features.shape
    bm = 400
    assert n % bm == 0
    bias2 = bias.reshape(1, d)

    support = pl.pallas_call(
        _support_body,
        in_specs=[
            pl.BlockSpec((n, d), lambda: (0, 0)),
            pl.BlockSpec((d, d), lambda: (0, 0)),
        ],
        out_specs=pl.BlockSpec((n, d), lambda: (0, 0)),
        out_shape=jax.ShapeDtypeStruct((n, d), jnp.float32),
    )(features, weight)

    return pl.pallas_call(
        _agg_body,
        grid=(n // bm,),
        in_specs=[
            pl.BlockSpec((bm, n), lambda i: (i, 0)),
            pl.BlockSpec((n, d), lambda i: (0, 0)),
            pl.BlockSpec((bm, d), lambda i: (i, 0)),
            pl.BlockSpec((1, d), lambda i: (0, 0)),
        ],
        out_specs=pl.BlockSpec((bm, d), lambda i: (i, 0)),
        out_shape=jax.ShapeDtypeStruct((n, d), jnp.float32),
        compiler_params=pltpu.CompilerParams(
            dimension_semantics=("parallel",),
        ),
    )(adj_norm, support, features, bias2)


# speedup vs baseline: 1.0739x; 1.0739x over previous
import jax
import jax.numpy as jnp
from jax.experimental import pallas as pl
from jax.experimental.pallas import tpu as pltpu


def _gcn_body(feat_ref, adj_ref, w_ref, b_ref, out_ref, support_ref):
    i = pl.program_id(0)
    bm = out_ref.shape[0]

    @pl.when(i == 0)
    def _():
        support_ref[...] = jnp.dot(
            feat_ref[...], w_ref[...], preferred_element_type=jnp.float32
        )

    acc = jnp.dot(adj_ref[...], support_ref[...], preferred_element_type=jnp.float32)
    feat_blk = feat_ref[pl.ds(i * bm, bm), :]
    out_ref[...] = jnp.maximum(acc + b_ref[...], 0.0) + feat_blk


def kernel(features, adj_norm, weight, bias):
    n, d = features.shape
    bm = 200
    assert n % bm == 0
    bias2 = bias.reshape(1, d)

    return pl.pallas_call(
        _gcn_body,
        grid=(n // bm,),
        in_specs=[
            pl.BlockSpec((n, d), lambda i: (0, 0)),
            pl.BlockSpec((bm, n), lambda i: (i, 0)),
            pl.BlockSpec((d, d), lambda i: (0, 0)),
            pl.BlockSpec((1, d), lambda i: (0, 0)),
        ],
        out_specs=pl.BlockSpec((bm, d), lambda i: (i, 0)),
        out_shape=jax.ShapeDtypeStruct((n, d), jnp.float32),
        scratch_shapes=[pltpu.VMEM((n, d), jnp.float32)],
    )(features, adj_norm, weight, bias2)


# speedup vs baseline: 1.0815x; 1.0071x over previous
import jax
import jax.numpy as jnp
from jax.experimental import pallas as pl
from jax.experimental.pallas import tpu as pltpu


def _gcn_body(feat_ref, adj_ref, w_ref, b_ref, out_ref, support_ref):
    i = pl.program_id(0)
    bm = out_ref.shape[0]

    @pl.when(i == 0)
    def _():
        support_ref[...] = jnp.dot(
            feat_ref[...], w_ref[...], preferred_element_type=jnp.float32
        )

    acc = jnp.dot(adj_ref[...], support_ref[...], preferred_element_type=jnp.float32)
    feat_blk = feat_ref[pl.ds(i * bm, bm), :]
    out_ref[...] = jnp.maximum(acc + b_ref[...], 0.0) + feat_blk


def kernel(features, adj_norm, weight, bias):
    n, d = features.shape
    bm = 400
    assert n % bm == 0
    bias2 = bias.reshape(1, d)

    return pl.pallas_call(
        _gcn_body,
        grid=(n // bm,),
        in_specs=[
            pl.BlockSpec((n, d), lambda i: (0, 0)),
            pl.BlockSpec((bm, n), lambda i: (i, 0)),
            pl.BlockSpec((d, d), lambda i: (0, 0)),
            pl.BlockSpec((1, d), lambda i: (0, 0)),
        ],
        out_specs=pl.BlockSpec((bm, d), lambda i: (i, 0)),
        out_shape=jax.ShapeDtypeStruct((n, d), jnp.float32),
        scratch_shapes=[pltpu.VMEM((n, d), jnp.float32)],
    )(features, adj_norm, weight, bias2)
